# Initial kernel scaffold; baseline (speedup 1.0000x reference)
import jax
import jax.numpy as jnp
from jax import lax
from jax.experimental import pallas as pl
from jax.experimental.pallas import tpu as pltpu
from jax.experimental.pallas import tpu_sc as plsc

_NC = 2
_NS = 16
_NW = _NC * _NS

_W = 224
_OW = 2 * _W
_R = 2 * 96 * 224
_RPW = _R // _NW
_K = 48
_NCH = _RPW // _K
_GRPS = _W // 16


def _body(img_hbm, out_hbm, in_buf, out_buf):
    c = lax.axis_index("c")
    s = lax.axis_index("s")
    wid = s * _NC + c
    base = wid * _RPW

    zero = jnp.zeros((16,), jnp.float32)

    @pl.loop(0, (_K * 2 * _OW) // 16)
    def _zero(i):
        out_buf[pl.ds(i * 16, 16)] = zero

    iota2 = lax.iota(jnp.int32, 16) * 2

    @pl.loop(0, _NCH)
    def _chunk(g):
        row0 = base + g * _K
        pltpu.sync_copy(img_hbm.at[pl.ds(row0 * _W, _K * _W)], in_buf)

        @pl.loop(0, _K)
        def _row(r):
            in_base = r * _W
            out_base = r * (2 * _OW)
            for j in range(_GRPS):
                x = in_buf[pl.ds(in_base + j * 16, 16)]
                idx = iota2 + (out_base + j * 32)
                plsc.store_scatter(out_buf, [idx], x)

        pltpu.sync_copy(out_buf, out_hbm.at[pl.ds(row0 * 2 * _OW, _K * 2 * _OW)])


def _make_kernel():
    mesh = plsc.VectorSubcoreMesh(
        core_axis_name="c", subcore_axis_name="s",
        num_cores=_NC, num_subcores=_NS)
    return pl.kernel(
        _body,
        out_type=jax.ShapeDtypeStruct((_R * 2 * _OW,), jnp.float32),
        mesh=mesh,
        scratch_types=[
            pltpu.VMEM((_K * _W,), jnp.float32),
            pltpu.VMEM((_K * 2 * _OW,), jnp.float32),
        ],
    )


_upsample = _make_kernel()


def kernel(img, scale_factor):
    B, C, H, W = img.shape
    out_flat = _upsample(img.reshape(B * C * H * W))
    return out_flat.reshape(B, C, 2 * H, 2 * W)


# speedup vs baseline: 1.2610x; 1.2610x over previous
import jax
import jax.numpy as jnp
from jax import lax
from jax.experimental import pallas as pl
from jax.experimental.pallas import tpu as pltpu
from jax.experimental.pallas import tpu_sc as plsc

_NC = 2
_NS = 16
_NW = _NC * _NS

_W = 224
_OW = 2 * _W
_R = 2 * 96 * 224
_RPW = _R // _NW
_K = 48
_NCH = _RPW // _K
_GRPS = _W // 16


def _body(img_hbm, out_hbm, in_buf, out_buf):
    c = lax.axis_index("c")
    s = lax.axis_index("s")
    wid = s * _NC + c
    base = wid * _RPW

    zero = jnp.zeros((16,), jnp.float32)

    @pl.loop(0, (_K * 2 * _OW) // 16)
    def _zero(i):
        out_buf[pl.ds(i * 16, 16)] = zero

    iota2 = lax.iota(jnp.int32, 16) * 2

    @pl.loop(0, _NCH)
    def _chunk(g):
        row0 = base + g * _K
        pltpu.sync_copy(img_hbm.at[pl.ds(row0 * _W, _K * _W)], in_buf)

        @pl.loop(0, _K)
        def _row(r):
            in_base = r * _W
            out_base = r * (2 * _OW)
            for j in range(_GRPS):
                x = in_buf[pl.ds(in_base + j * 16, 16)]
                idx = iota2 + (out_base + j * 32)
                plsc.store_scatter(out_buf, [idx], x)

        pltpu.sync_copy(out_buf, out_hbm.at[pl.ds(row0 * 2 * _OW, _K * 2 * _OW)])


import functools


@functools.cache
def _make_kernel():
    mesh = plsc.VectorSubcoreMesh(
        core_axis_name="c", subcore_axis_name="s",
        num_cores=_NC, num_subcores=_NS)
    return pl.kernel(
        _body,
        out_type=jax.ShapeDtypeStruct((_R * 2 * _OW,), jnp.float32),
        mesh=mesh,
        scratch_types=[
            pltpu.VMEM((_K * _W,), jnp.float32),
            pltpu.VMEM((_K * 2 * _OW,), jnp.float32),
        ],
        compiler_params=pltpu.CompilerParams(
            needs_layout_passes=False,
            use_tc_tiling_on_sc=False,
        ),
    )


def kernel(img, scale_factor):
    B, C, H, W = img.shape
    out_flat = _make_kernel()(img.reshape(B * C * H * W))
    return out_flat.reshape(B, C, 2 * H, 2 * W)


# speedup vs baseline: 1.5173x; 1.2033x over previous
import jax
import jax.numpy as jnp
from jax import lax
from jax.experimental import pallas as pl
from jax.experimental.pallas import tpu as pltpu
from jax.experimental.pallas import tpu_sc as plsc

_NC = 2
_NS = 16
_NW = _NC * _NS

_W = 224
_OW = 2 * _W
_R = 2 * 96 * 224
_RPW = _R // _NW
_K = 48
_NCH = _RPW // _K
_GRPS = _W // 16


def _body(img_hbm, out_hbm, in0, in1, out0, out1,
          sem_in0, sem_in1, sem_out0, sem_out1):
    c = lax.axis_index("c")
    s = lax.axis_index("s")
    wid = s * _NC + c
    base = wid * _RPW

    ins = (in0, in1)
    outs = (out0, out1)
    sem_ins = (sem_in0, sem_in1)
    sem_outs = (sem_out0, sem_out1)

    zero = jnp.zeros((16,), jnp.float32)

    for ob in outs:
        @pl.loop(0, (_K * 2 * _OW) // 16)
        def _zero(i, ob=ob):
            ob[pl.ds(i * 16, 16)] = zero

    iota2 = lax.iota(jnp.int32, 16) * 2

    def in_slice(q):
        return img_hbm.at[pl.ds((base + q * _K) * _W, _K * _W)]

    def out_slice(q):
        return out_hbm.at[pl.ds((base + q * _K) * 2 * _OW, _K * 2 * _OW)]

    def compute(ib, ob):
        @pl.loop(0, _K)
        def _row(r):
            in_base = r * _W
            row_ref = ob.at[pl.ds(r * (2 * _OW), 2 * _OW)]
            for j in range(_GRPS):
                x = ib[pl.ds(in_base + j * 16, 16)]
                plsc.store_scatter(row_ref, [iota2 + j * 32], x)

    pltpu.async_copy(in_slice(0), ins[0], sem_ins[0])

    @pl.loop(0, _NCH, step=2)
    def _chunks(g):
        for b in range(2):
            q = g + b

            @pl.when(q + 1 < _NCH)
            def _prefetch():
                pltpu.async_copy(in_slice(q + 1), ins[1 - b], sem_ins[1 - b])

            pltpu.make_async_copy(in_slice(q), ins[b], sem_ins[b]).wait()

            @pl.when(q >= 2)
            def _drain_prev():
                pltpu.make_async_copy(outs[b], out_slice(q), sem_outs[b]).wait()

            compute(ins[b], outs[b])
            pltpu.async_copy(outs[b], out_slice(q), sem_outs[b])

    for b in range(2):
        pltpu.make_async_copy(outs[b], out_slice(_NCH - 2 + b), sem_outs[b]).wait()


import functools


@functools.cache
def _make_kernel():
    mesh = plsc.VectorSubcoreMesh(
        core_axis_name="c", subcore_axis_name="s",
        num_cores=_NC, num_subcores=_NS)
    return pl.kernel(
        _body,
        out_type=jax.ShapeDtypeStruct((_R * 2 * _OW,), jnp.float32),
        mesh=mesh,
        scratch_types=[
            pltpu.VMEM((_K * _W,), jnp.float32),
            pltpu.VMEM((_K * _W,), jnp.float32),
            pltpu.VMEM((_K * 2 * _OW,), jnp.float32),
            pltpu.VMEM((_K * 2 * _OW,), jnp.float32),
            pltpu.SemaphoreType.DMA,
            pltpu.SemaphoreType.DMA,
            pltpu.SemaphoreType.DMA,
            pltpu.SemaphoreType.DMA,
        ],
        compiler_params=pltpu.CompilerParams(
            needs_layout_passes=False,
            use_tc_tiling_on_sc=False,
        ),
    )


def kernel(img, scale_factor):
    B, C, H, W = img.shape
    out_flat = _make_kernel()(img.reshape(B * C * H * W))
    return out_flat.reshape(B, C, 2 * H, 2 * W)


# speedup vs baseline: 2.6695x; 1.7594x over previous
import functools

import jax
import jax.numpy as jnp
from jax import lax
from jax.experimental import pallas as pl
from jax.experimental.pallas import tpu as pltpu
from jax.experimental.pallas import tpu_sc as plsc

_NC = 2
_NS = 16
_NW = _NC * _NS

_R = 2 * 96 * 224
_W = 224
_OW = 2 * _W
_RPW = _R // _NW
_K = 32
_NCH = _RPW // _K
_GRPS = _W // 16


def _body(img_hbm, out_hbm, in0, in1, out0, out1,
          sem_in0, sem_in1, sem_out0, sem_out1):
    c = lax.axis_index("c")
    s = lax.axis_index("s")
    wid = s * _NC + c
    row0 = wid * _RPW

    ins = (in0, in1)
    outs = (out0, out1)
    sem_ins = (sem_in0, sem_in1)
    sem_outs = (sem_out0, sem_out1)

    zero = jnp.zeros((16,), jnp.float32)

    for ob in outs:
        @pl.loop(0, 2 * _K)
        def _zero_row(r, ob=ob):
            for k in range(_OW // 16):
                ob[r, pl.ds(k * 16, 16)] = zero

    iota2 = lax.iota(jnp.int32, 16) * 2

    def in_slice(q):
        return img_hbm.at[pl.ds(row0 + q * _K, _K), :]

    def out_slice(q):
        return out_hbm.at[pl.ds(2 * (row0 + q * _K), 2 * _K), :]

    def compute(ib, ob):
        @pl.loop(0, _K)
        def _row(r):
            rvec = jnp.zeros((16,), jnp.int32) + 2 * r
            for j in range(_GRPS):
                x = ib[r, pl.ds(j * 16, 16)]
                plsc.store_scatter(ob, [rvec, iota2 + j * 32], x)

    @pl.loop(0, _NCH)
    def _chunks(q):
        pltpu.sync_copy(in_slice(q), ins[0])
        compute(ins[0], outs[0])
        pltpu.sync_copy(outs[0], out_slice(q))


@functools.cache
def _make_kernel():
    mesh = plsc.VectorSubcoreMesh(
        core_axis_name="c", subcore_axis_name="s",
        num_cores=_NC, num_subcores=_NS)
    return pl.kernel(
        _body,
        out_type=jax.ShapeDtypeStruct((2 * _R, _OW), jnp.float32),
        mesh=mesh,
        scratch_types=[
            pltpu.VMEM((_K, _W), jnp.float32),
            pltpu.VMEM((_K, _W), jnp.float32),
            pltpu.VMEM((2 * _K, _OW), jnp.float32),
            pltpu.VMEM((2 * _K, _OW), jnp.float32),
            pltpu.SemaphoreType.DMA,
            pltpu.SemaphoreType.DMA,
            pltpu.SemaphoreType.DMA,
            pltpu.SemaphoreType.DMA,
        ],
        compiler_params=pltpu.CompilerParams(
            needs_layout_passes=False,
        ),
    )


def kernel(img, scale_factor):
    B, C, H, W = img.shape
    out2 = _make_kernel()(img.reshape(B * C * H, W))
    return out2.reshape(B, C, 2 * H, 2 * W)
